# baseline (speedup 1.0000x reference)
import functools
import math

import jax
import jax.numpy as jnp
from jax import lax
from jax.experimental import pallas as pl
from jax.experimental.pallas import tpu as pltpu
from jax.experimental.pallas import tpu_sc as plsc

_VOCAB = 1000000
_EMBED = 128
_MAX_LEN = 512
_B, _L = 1024, 200

_NC, _NS = 2, 16
_NW = _NC * _NS
_ROWS = _B * _L
_RPW = _ROWS // _NW
_CHUNK = 128
_NCHUNK = _RPW // _CHUNK
_NBUF = 3
_AHEAD = 2


def _pos_table():
    position = jnp.arange(_MAX_LEN, dtype=jnp.float32)[:, None]
    div_term = jnp.exp(
        jnp.arange(0, _EMBED, 2, dtype=jnp.float32) * -(math.log(10000.0) / _EMBED)
    )
    pe = jnp.zeros((_MAX_LEN, _EMBED), dtype=jnp.float32)
    pe = pe.at[:, 0::2].set(jnp.sin(position * div_term))
    pe = pe.at[:, 1::2].set(jnp.cos(position * div_term))
    return pe[:_L]


_mesh = plsc.VectorSubcoreMesh(core_axis_name="c", subcore_axis_name="s")


@functools.partial(
    pl.kernel,
    out_type=jax.ShapeDtypeStruct((_ROWS, _EMBED), jnp.float32),
    mesh=_mesh,
    scratch_types=(
        [pltpu.VMEM((_NCHUNK, _CHUNK), jnp.int32),
         pltpu.VMEM((2 * _L, _EMBED), jnp.float32)]
        + [pltpu.VMEM((_CHUNK, _EMBED), jnp.float32) for _ in range(_NBUF)]
        + [pltpu.SemaphoreType.DMA for _ in range(2 * _NBUF)]
    ),
)
def _embed_kernel(table_hbm, idx_hbm, pe_hbm, out_hbm, idx_v, pe_v, *rest):
    bufs = rest[:_NBUF]
    sems_i = rest[_NBUF:2 * _NBUF]
    sems_o = rest[2 * _NBUF:]

    wid = lax.axis_index("s") * _NC + lax.axis_index("c")
    base = wid * _RPW
    pltpu.sync_copy(idx_hbm.at[wid], idx_v)
    pltpu.sync_copy(pe_hbm, pe_v)

    def gather_chunk(c, buf, sem):
        pltpu.async_copy(table_hbm.at[idx_v.at[c]], buf, sem)

    def wait_gather(buf, sem):
        pltpu.make_async_copy(table_hbm.at[idx_v.at[0]], buf, sem).wait()

    def issue_out(c, buf, sem):
        pltpu.async_copy(buf, out_hbm.at[pl.ds(base + c * _CHUNK, _CHUNK)], sem)

    def wait_out(buf, sem):
        pltpu.make_async_copy(buf, out_hbm.at[pl.ds(base, _CHUNK)], sem).wait()

    def add_pe(buf, poff):
        def row_body(i, carry2):
            prow = poff + i
            for j in range(_EMBED // 16):
                cols = pl.ds(j * 16, 16)
                plsc.addupdate(buf.at[i, cols], pe_v[prow, cols])
            return carry2

        lax.fori_loop(0, _CHUNK, row_body, 0, unroll=2)

    def on_buf(sel, fn):
        for k in range(_NBUF):
            @pl.when(sel == k)
            def _(k=k):
                fn(bufs[k], sems_i[k], sems_o[k])

    for k in range(_AHEAD):
        gather_chunk(k, bufs[k], sems_i[k])

    def body(t, carry):
        def process(buf, sem_i, sem_o):
            wait_gather(buf, sem_i)
            add_pe(buf, lax.rem(t * _CHUNK, _L))
            issue_out(t, buf, sem_o)

        on_buf(lax.rem(t, _NBUF), process)

        @pl.when(t + _AHEAD < _NCHUNK)
        def _():
            def prefetch(buf, sem_i, sem_o):
                @pl.when(t >= _NBUF - _AHEAD)
                def _():
                    wait_out(buf, sem_o)
                gather_chunk(t + _AHEAD, buf, sem_i)

            on_buf(lax.rem(t + _AHEAD, _NBUF), prefetch)

        return carry

    lax.fori_loop(0, _NCHUNK, body, 0)
    for k in range(_NBUF):
        wait_out(bufs[k], sems_o[k])


def kernel(sequence, token_table):
    idx = sequence.astype(jnp.int32).reshape(_NW, _NCHUNK, _CHUNK)
    pe = _pos_table()
    pe2 = jnp.concatenate([pe, pe], axis=0)
    out = _embed_kernel(token_table, idx, pe2)
    return out.reshape(_B, _L, _EMBED)


# speedup vs baseline: 2.3338x; 2.3338x over previous
import functools
import math

import jax
import jax.numpy as jnp
from jax import lax
from jax.experimental import pallas as pl
from jax.experimental.pallas import tpu as pltpu
from jax.experimental.pallas import tpu_sc as plsc

_VOCAB = 1000000
_EMBED = 128
_MAX_LEN = 512
_B, _L = 1024, 200

_NC, _NS = 2, 16
_NW = _NC * _NS
_ROWS = _B * _L
_RPW = _ROWS // _NW
_CHUNK = 200
_GSPLIT = 2
_SUB = _CHUNK // _GSPLIT
_NCHUNK = _RPW // _CHUNK
_NBUF = 3
_AHEAD = 2

_PE_ROWS = _CHUNK if _CHUNK % _L == 0 else _L * (1 + -(-_CHUNK // _L))


def _pos_table():
    position = jnp.arange(_MAX_LEN, dtype=jnp.float32)[:, None]
    div_term = jnp.exp(
        jnp.arange(0, _EMBED, 2, dtype=jnp.float32) * -(math.log(10000.0) / _EMBED)
    )
    pe = jnp.zeros((_MAX_LEN, _EMBED), dtype=jnp.float32)
    pe = pe.at[:, 0::2].set(jnp.sin(position * div_term))
    pe = pe.at[:, 1::2].set(jnp.cos(position * div_term))
    return pe[:_L]


_mesh = plsc.VectorSubcoreMesh(core_axis_name="c", subcore_axis_name="s")


@functools.partial(
    pl.kernel,
    out_type=jax.ShapeDtypeStruct((_ROWS, _EMBED), jnp.float32),
    mesh=_mesh,
    scratch_types=(
        [pltpu.VMEM((_NCHUNK, _GSPLIT, _SUB), jnp.int32),
         pltpu.VMEM((_PE_ROWS, _EMBED), jnp.float32)]
        + [pltpu.VMEM((_CHUNK, _EMBED), jnp.float32) for _ in range(_NBUF)]
        + [pltpu.SemaphoreType.DMA for _ in range(2 * _NBUF)]
    ),
)
def _embed_kernel(table_hbm, idx_hbm, pe_hbm, out_hbm, idx_v, pe_v, *rest):
    bufs = rest[:_NBUF]
    sems_i = rest[_NBUF:2 * _NBUF]
    sems_o = rest[2 * _NBUF:]

    wid = lax.axis_index("s") * _NC + lax.axis_index("c")
    base = wid * _RPW
    pltpu.sync_copy(idx_hbm.at[wid], idx_v)
    pltpu.sync_copy(pe_hbm, pe_v)

    def gather_chunk(c, buf, sem):
        for g in range(_GSPLIT):
            pltpu.async_copy(
                table_hbm.at[idx_v.at[c, g]], buf.at[pl.ds(g * _SUB, _SUB)], sem)

    def wait_gather(buf, sem):
        for g in range(_GSPLIT):
            pltpu.make_async_copy(
                table_hbm.at[idx_v.at[0, 0]],
                buf.at[pl.ds(g * _SUB, _SUB)], sem).wait()

    def issue_out(c, buf, sem):
        pltpu.async_copy(buf, out_hbm.at[pl.ds(base + c * _CHUNK, _CHUNK)], sem)

    def wait_out(buf, sem):
        pltpu.make_async_copy(buf, out_hbm.at[pl.ds(base, _CHUNK)], sem).wait()

    def add_pe(buf, poff):
        def row_body(i, carry2):
            prow = poff + i
            for j in range(_EMBED // 16):
                cols = pl.ds(j * 16, 16)
                plsc.addupdate(buf.at[i, cols], pe_v[prow, cols])
            return carry2

        lax.fori_loop(0, _CHUNK, row_body, 0, unroll=2)

    def on_buf(sel, fn):
        for k in range(_NBUF):
            @pl.when(sel == k)
            def _(k=k):
                fn(bufs[k], sems_i[k], sems_o[k])

    for k in range(_AHEAD):
        gather_chunk(k, bufs[k], sems_i[k])

    def body(t, carry):
        def process(buf, sem_i, sem_o):
            wait_gather(buf, sem_i)
            if _CHUNK % _L == 0:
                add_pe(buf, 0)
            else:
                add_pe(buf, lax.rem(t * _CHUNK, _L))
            issue_out(t, buf, sem_o)

        on_buf(lax.rem(t, _NBUF), process)

        @pl.when(t + _AHEAD < _NCHUNK)
        def _():
            def prefetch(buf, sem_i, sem_o):
                @pl.when(t >= _NBUF - _AHEAD)
                def _():
                    wait_out(buf, sem_o)
                gather_chunk(t + _AHEAD, buf, sem_i)

            on_buf(lax.rem(t + _AHEAD, _NBUF), prefetch)

        return carry

    lax.fori_loop(0, _NCHUNK, body, 0)
    for k in range(_NBUF):
        wait_out(bufs[k], sems_o[k])


def kernel(sequence, token_table):
    idx = sequence.astype(jnp.int32).reshape(_NW, _NCHUNK, _GSPLIT, _SUB)
    pe = _pos_table()
    reps = -(-_PE_ROWS // _L)
    pe_full = jnp.concatenate([pe] * reps, axis=0)[:_PE_ROWS]
    out = _embed_kernel(token_table, idx, pe_full)
    return out.reshape(_B, _L, _EMBED)
